# Initial kernel scaffold; baseline (speedup 1.0000x reference)
import jax
import jax.numpy as jnp
from jax.experimental import pallas as pl


def kernel(history_stack, edge_index, W_enc, b_enc, W_gat, attn_l, attn_r, b_gat, W_ih, W_hh, b_ih, b_hh, W_dec, b_dec):
    raise NotImplementedError("write your pallas kernel here")


# speedup vs baseline: 79.6654x; 79.6654x over previous
import jax
import jax.numpy as jnp
from jax.experimental import pallas as pl
from jax.experimental.pallas import tpu as pltpu

B, T, F, H, HEADS = 16, 20, 64, 128, 4
OUT = H // HEADS
E0 = 512


def _gat_kernel(hist_ref, ei_ref, W_enc_ref, b_enc_ref, W_gat_ref,
                al_ref, ar_ref, bgat_ref, out_ref):
    hist = hist_ref[0]
    x = hist * W_enc_ref[...][None] + b_enc_ref[...][None]
    feat = jax.lax.dot_general(x, W_gat_ref[...],
                               (((2,), (0,)), ((), ())),
                               preferred_element_type=jnp.float32)

    src = ei_ref[0]
    dst = ei_ref[1]
    ids_e = jax.lax.broadcasted_iota(jnp.int32, (F, E0), 0)
    m_src = (ids_e == src[None, :]).astype(jnp.float32)
    m_dst = (ids_e == dst[None, :]).astype(jnp.float32)
    A = jax.lax.dot_general(m_dst, m_src, (((1,), (1,)), ((), ())),
                            preferred_element_type=jnp.float32)
    r_i = jax.lax.broadcasted_iota(jnp.int32, (F, F), 0)
    c_i = jax.lax.broadcasted_iota(jnp.int32, (F, F), 1)
    A = A + (r_i == c_i).astype(jnp.float32)
    pos = A > 0.0

    al = al_ref[...]
    ar = ar_ref[...]
    outs = []
    for h in range(HEADS):
        fh = feat[:, :, h * OUT:(h + 1) * OUT]
        er_col = (fh * ar[h][None, None, :]).sum(-1, keepdims=True)
        el_row = (fh * al[h][None, None, :]).sum(-1)
        e = er_col + el_row[:, None, :]
        e = jnp.where(e >= 0, e, 0.2 * e)
        e = jnp.where(pos[None], e, -1e30)
        m = e.max(axis=-1, keepdims=True)
        ee = A[None] * jnp.exp(e - m)
        s = ee.sum(axis=-1, keepdims=True)
        w = ee / s
        outs.append(jax.lax.dot_general(
            w, fh, (((2,), (1,)), ((0,), (0,))),
            preferred_element_type=jnp.float32))
    rst = jnp.concatenate(outs, axis=-1)
    out_ref[...] = rst + bgat_ref[...][None, None]


def _gru_kernel(gnn_ref, Wih_ref, Whh_ref, bih_ref, bhh_ref, Wdec_ref,
                hid_ref, ans_ref):
    x_all = gnn_ref[0]
    Wih = Wih_ref[0]
    Whh = Whh_ref[0]
    gi_all = jax.lax.dot_general(x_all, Wih, (((1,), (1,)), ((), ())),
                                 preferred_element_type=jnp.float32)
    gi_all = gi_all + bih_ref[0, 0][None]
    bhh = bhh_ref[0, 0]
    wdec = Wdec_ref[0, 0]

    h = jnp.zeros((B, H), jnp.float32)
    for t in range(T):
        gi = gi_all[t * B:(t + 1) * B]
        gh = jax.lax.dot_general(h, Whh, (((1,), (1,)), ((), ())),
                                 preferred_element_type=jnp.float32)
        gh = gh + bhh[None]
        r = jax.nn.sigmoid(gi[:, :H] + gh[:, :H])
        z = jax.nn.sigmoid(gi[:, H:2 * H] + gh[:, H:2 * H])
        n = jnp.tanh(gi[:, 2 * H:] + r * gh[:, 2 * H:])
        h = (1.0 - z) * n + z * h
        hid_ref[0, t * B:(t + 1) * B, :] = h
        ans_ref[0, t, :] = (h * wdec[None]).sum(-1)


def kernel(history_stack, edge_index, W_enc, b_enc, W_gat, attn_l, attn_r,
           b_gat, W_ih, W_hh, b_ih, b_hh, W_dec, b_dec):
    hist_t = history_stack.transpose(1, 0, 2)[..., None]
    ei = edge_index.astype(jnp.int32)

    gnn = pl.pallas_call(
        _gat_kernel,
        grid=(T,),
        in_specs=[
            pl.BlockSpec((1, B, F, 1), lambda t: (t, 0, 0, 0)),
            pl.BlockSpec((2, E0), lambda t: (0, 0)),
            pl.BlockSpec((F, H), lambda t: (0, 0)),
            pl.BlockSpec((F, H), lambda t: (0, 0)),
            pl.BlockSpec((H, H), lambda t: (0, 0)),
            pl.BlockSpec((HEADS, OUT), lambda t: (0, 0)),
            pl.BlockSpec((HEADS, OUT), lambda t: (0, 0)),
            pl.BlockSpec((H,), lambda t: (0,)),
        ],
        out_specs=pl.BlockSpec((B, F, H), lambda t: (t, 0, 0)),
        out_shape=jax.ShapeDtypeStruct((T * B, F, H), jnp.float32),
        compiler_params=pltpu.CompilerParams(
            dimension_semantics=("parallel",)),
    )(hist_t, ei, W_enc, b_enc, W_gat, attn_l, attn_r, b_gat)

    gnn_perm = gnn.transpose(1, 0, 2)

    hid_perm, ans_perm = pl.pallas_call(
        _gru_kernel,
        grid=(F,),
        in_specs=[
            pl.BlockSpec((1, T * B, H), lambda f: (f, 0, 0)),
            pl.BlockSpec((1, 3 * H, H), lambda f: (f, 0, 0)),
            pl.BlockSpec((1, 3 * H, H), lambda f: (f, 0, 0)),
            pl.BlockSpec((1, 1, 3 * H), lambda f: (f, 0, 0)),
            pl.BlockSpec((1, 1, 3 * H), lambda f: (f, 0, 0)),
            pl.BlockSpec((1, 1, H), lambda f: (f, 0, 0)),
        ],
        out_specs=[
            pl.BlockSpec((1, T * B, H), lambda f: (f, 0, 0)),
            pl.BlockSpec((1, T, B), lambda f: (f, 0, 0)),
        ],
        out_shape=[
            jax.ShapeDtypeStruct((F, T * B, H), jnp.float32),
            jax.ShapeDtypeStruct((F, T, B), jnp.float32),
        ],
        compiler_params=pltpu.CompilerParams(
            dimension_semantics=("parallel",)),
    )(gnn_perm, W_ih, W_hh, b_ih[:, None], b_hh[:, None], W_dec[:, None])

    hiddens = hid_perm.reshape(F, T, B, H).transpose(2, 1, 0, 3)
    ans = ans_perm.transpose(2, 1, 0) + b_dec[None, None]
    return (ans, hiddens)


# speedup vs baseline: 133.6537x; 1.6777x over previous
import jax
import jax.numpy as jnp
from jax.experimental import pallas as pl
from jax.experimental.pallas import tpu as pltpu

B, T, F, H, HEADS = 16, 20, 64, 128, 4
OUT = H // HEADS
E0 = 512
FG = 8


def _adj_kernel(ei_ref, a_ref):
    src = ei_ref[0]
    dst = ei_ref[1]
    ids_e = jax.lax.broadcasted_iota(jnp.int32, (F, E0), 0)
    m_src = (ids_e == src[None, :]).astype(jnp.float32)
    m_dst = (ids_e == dst[None, :]).astype(jnp.float32)
    A = jax.lax.dot_general(m_dst, m_src, (((1,), (1,)), ((), ())),
                            preferred_element_type=jnp.float32)
    r_i = jax.lax.broadcasted_iota(jnp.int32, (F, F), 0)
    c_i = jax.lax.broadcasted_iota(jnp.int32, (F, F), 1)
    a_ref[...] = A + (r_i == c_i).astype(jnp.float32)


def _gat_kernel(hist_ref, a_ref, W_enc_ref, b_enc_ref, W_gat_ref,
                al_ref, ar_ref, bgat_ref, out_ref):
    hist = hist_ref[0]
    x = hist * W_enc_ref[...][None] + b_enc_ref[...][None]
    feat = jax.lax.dot_general(x, W_gat_ref[...],
                               (((2,), (0,)), ((), ())),
                               preferred_element_type=jnp.float32)
    A = a_ref[...]
    pos = A > 0.0

    al = al_ref[...]
    ar = ar_ref[...]
    outs = []
    for h in range(HEADS):
        fh = feat[:, :, h * OUT:(h + 1) * OUT]
        er_col = (fh * ar[h][None, None, :]).sum(-1, keepdims=True)
        el_row = (fh * al[h][None, None, :]).sum(-1)
        e = er_col + el_row[:, None, :]
        e = jnp.where(e >= 0, e, 0.2 * e)
        e = jnp.where(pos[None], e, -1e30)
        m = e.max(axis=-1, keepdims=True)
        ee = A[None] * jnp.exp(e - m)
        s = ee.sum(axis=-1, keepdims=True)
        w = ee / s
        outs.append(jax.lax.dot_general(
            w, fh, (((2,), (1,)), ((0,), (0,))),
            preferred_element_type=jnp.float32))
    rst = jnp.concatenate(outs, axis=-1)
    out_ref[:, 0, :, :] = rst + bgat_ref[...][None, None]


def _gru_kernel(gnn_ref, Wih_ref, Whh_ref, bih_ref, bhh_ref, Wdec_ref,
                hid_ref, ans_ref):
    gis = []
    hs = []
    for fl in range(FG):
        x_f = gnn_ref[:, :, fl, :]
        gi = jax.lax.dot_general(x_f, Wih_ref[fl], (((2,), (1,)), ((), ())),
                                 preferred_element_type=jnp.float32)
        gis.append(gi + bih_ref[fl, 0][None, None])
        hs.append(jnp.zeros((B, H), jnp.float32))

    for t in range(T):
        for fl in range(FG):
            h = hs[fl]
            gi = gis[fl][:, t, :]
            gh = jax.lax.dot_general(h, Whh_ref[fl], (((1,), (1,)), ((), ())),
                                     preferred_element_type=jnp.float32)
            gh = gh + bhh_ref[fl, 0][None]
            r = jax.nn.sigmoid(gi[:, :H] + gh[:, :H])
            z = jax.nn.sigmoid(gi[:, H:2 * H] + gh[:, H:2 * H])
            n = jnp.tanh(gi[:, 2 * H:] + r * gh[:, 2 * H:])
            h = (1.0 - z) * n + z * h
            hs[fl] = h
            hid_ref[:, t, fl, :] = h
            ans_ref[fl, t, :] = (h * Wdec_ref[fl, 0][None]).sum(-1)


def kernel(history_stack, edge_index, W_enc, b_enc, W_gat, attn_l, attn_r,
           b_gat, W_ih, W_hh, b_ih, b_hh, W_dec, b_dec):
    hist_t = history_stack.transpose(1, 0, 2)[..., None]
    ei = edge_index.astype(jnp.int32)

    A = pl.pallas_call(
        _adj_kernel,
        out_shape=jax.ShapeDtypeStruct((F, F), jnp.float32),
    )(ei)

    gnn = pl.pallas_call(
        _gat_kernel,
        grid=(T,),
        in_specs=[
            pl.BlockSpec((1, B, F, 1), lambda t: (t, 0, 0, 0)),
            pl.BlockSpec((F, F), lambda t: (0, 0)),
            pl.BlockSpec((F, H), lambda t: (0, 0)),
            pl.BlockSpec((F, H), lambda t: (0, 0)),
            pl.BlockSpec((H, H), lambda t: (0, 0)),
            pl.BlockSpec((HEADS, OUT), lambda t: (0, 0)),
            pl.BlockSpec((HEADS, OUT), lambda t: (0, 0)),
            pl.BlockSpec((H,), lambda t: (0,)),
        ],
        out_specs=pl.BlockSpec((B, 1, F, H), lambda t: (0, t, 0, 0)),
        out_shape=jax.ShapeDtypeStruct((B, T, F, H), jnp.float32),
        compiler_params=pltpu.CompilerParams(
            dimension_semantics=("parallel",)),
    )(hist_t, A, W_enc, b_enc, W_gat, attn_l, attn_r, b_gat)

    hiddens, ans_perm = pl.pallas_call(
        _gru_kernel,
        grid=(F // FG,),
        in_specs=[
            pl.BlockSpec((B, T, FG, H), lambda f: (0, 0, f, 0)),
            pl.BlockSpec((FG, 3 * H, H), lambda f: (f, 0, 0)),
            pl.BlockSpec((FG, 3 * H, H), lambda f: (f, 0, 0)),
            pl.BlockSpec((FG, 1, 3 * H), lambda f: (f, 0, 0)),
            pl.BlockSpec((FG, 1, 3 * H), lambda f: (f, 0, 0)),
            pl.BlockSpec((FG, 1, H), lambda f: (f, 0, 0)),
        ],
        out_specs=[
            pl.BlockSpec((B, T, FG, H), lambda f: (0, 0, f, 0)),
            pl.BlockSpec((FG, T, B), lambda f: (f, 0, 0)),
        ],
        out_shape=[
            jax.ShapeDtypeStruct((B, T, F, H), jnp.float32),
            jax.ShapeDtypeStruct((F, T, B), jnp.float32),
        ],
        compiler_params=pltpu.CompilerParams(
            dimension_semantics=("parallel",)),
    )(gnn, W_ih, W_hh, b_ih[:, None], b_hh[:, None], W_dec[:, None])

    ans = ans_perm.transpose(2, 1, 0) + b_dec[None, None]
    return (ans, hiddens)
